# baseline (speedup 1.0000x reference)
import functools

import jax
import jax.numpy as jnp
import numpy as np
from jax import lax
from jax.experimental import pallas as pl
from jax.experimental.pallas import tpu as pltpu
from jax.experimental.pallas import tpu_sc as plsc

_N = 10000
_E = 160000
_F = 128
_C = 16
_NB = 8
_MSG = 256
_AVG = 16.0

_PERM = np.empty(_MSG, np.int32)
for _j in range(_MSG):
    _c, _k = _j // 16, _j % 16
    _PERM[_j] = _c if _k == 0 else 16 + _c * 15 + (_k - 1)

_KMASK = (np.arange(16)[:, None] == (np.arange(_MSG)[None, :] % 16)).astype(
    np.float32)

_NC = 2
_NS = 16
_KB = 64
_EPAD = 163840
_EPT = _EPAD // _NS
_NBATCH = _EPT // _KB
_RPT = 624
_ZR = 16


def _pre_body(nf_ref, wup_ref, wproj_ref, hp_ref):
    h = jnp.dot(nf_ref[:], wup_ref[:], preferred_element_type=jnp.float32)
    h = h * lax.rsqrt(jnp.mean(h * h, axis=-1, keepdims=True) + 1e-6)
    hp_ref[:] = jnp.dot(h, wproj_ref[:], preferred_element_type=jnp.float32)


def _pre(nf, wup, wproj_exp):
    bn = 2000
    return pl.pallas_call(
        _pre_body,
        grid=(_N // bn,),
        in_specs=[
            pl.BlockSpec((bn, _F), lambda i: (i, 0)),
            pl.BlockSpec((_F, _F), lambda i: (0, 0)),
            pl.BlockSpec((_F, _MSG), lambda i: (0, 0)),
        ],
        out_specs=pl.BlockSpec((bn, _MSG), lambda i: (i, 0)),
        out_shape=jax.ShapeDtypeStruct((_N, _MSG), jnp.float32),
    )(nf, wup, wproj_exp)


def _edge_body(vt_ref, ret_ref, w2_ref, p_ref):
    x = vt_ref[0:1, :]
    y = vt_ref[1:2, :]
    z = vt_ref[2:3, :]
    r = lax.rsqrt(x * x + y * y + z * z + 1e-12)
    x = x * r
    y = y * r
    z = z * r
    s3 = 3.0 ** 0.5
    s5 = 5.0 ** 0.5
    s15 = 15.0 ** 0.5
    s358 = (35.0 / 8.0) ** 0.5
    s105 = 105.0 ** 0.5
    s218 = (21.0 / 8.0) ** 0.5
    s7 = 7.0 ** 0.5
    comps = [
        jnp.ones_like(x),
        s3 * x, s3 * y, s3 * z,
        s15 * x * y, s15 * y * z, (s5 / 2.0) * (3.0 * z * z - 1.0),
        s15 * x * z, (s15 / 2.0) * (x * x - y * y),
        s358 * y * (3.0 * x * x - y * y), s105 * x * y * z,
        s218 * y * (5.0 * z * z - 1.0),
        (s7 / 2.0) * (5.0 * z ** 3 - 3.0 * z),
        s218 * x * (5.0 * z * z - 1.0),
        (s105 / 2.0) * (x * x - y * y) * z,
        s358 * x * (x * x - 3.0 * y * y),
    ]
    be = x.shape[1]
    col = lax.broadcasted_iota(jnp.int32, (1, be), 1) + pl.program_id(0) * be
    valid = (col < _E).astype(jnp.float32)
    sT = jnp.concatenate(comps, axis=0) * valid
    blocks = [ret_ref[b:b + 1, :] * sT for b in range(_NB)]
    gT = jnp.concatenate(blocks + [sT], axis=0)
    p_ref[:] = lax.dot_general(
        gT, w2_ref[:], (((0,), (0,)), ((), ())),
        preferred_element_type=jnp.float32)


def _edge(vecT, reT, w2f):
    be = 2048
    return pl.pallas_call(
        _edge_body,
        grid=(_EPAD // be,),
        in_specs=[
            pl.BlockSpec((3, be), lambda i: (0, i)),
            pl.BlockSpec((_NB, be), lambda i: (0, i)),
            pl.BlockSpec((_NB * 16 + 16, _MSG), lambda i: (0, 0)),
        ],
        out_specs=pl.BlockSpec((be, _MSG), lambda i: (i, 0)),
        out_shape=jax.ShapeDtypeStruct((_EPAD, _MSG), jnp.float32),
    )(vecT, reT, w2f)


def _sc_body(hp_hbm, p_hbm, idx_hbm, out_hbm,
             agg_sh, packed, sb0, sb1, rb0, rb1, mr0, mr1, pr0, pr1, zbuf,
             sem0, sem1, ssem0, ssem1):
    cid = lax.axis_index("c")
    sid = lax.axis_index("s")
    is0 = cid == 0
    coff = cid * 128

    for rr in range(_ZR):
        for q in range(8):
            zbuf[rr, pl.ds(q * 16, 16)] = jnp.zeros((16,), jnp.float32)
    row0 = sid * _RPT
    for i in range(_RPT // _ZR):
        pltpu.sync_copy(zbuf, agg_sh.at[pl.ds(row0 + i * _ZR, _ZR)])

    @pl.when(sid == _NS - 1)
    def _zero_tail():
        pltpu.sync_copy(zbuf, agg_sh.at[pl.ds(row0 + _RPT, _ZR)])

    ebase = sid * _EPT
    pltpu.sync_copy(idx_hbm.at[pl.ds(ebase, _EPT)], packed)
    plsc.subcore_barrier()

    def fire(i, pr, mr, sb, sem):
        for k in range(_KB // 16):
            sb[pl.ds(k * 16, 16)] = lax.shift_right_logical(
                packed[pl.ds(i * _KB + k * 16, 16)], 16)
        e0 = ebase + i * _KB
        pltpu.async_copy(p_hbm.at[pl.ds(e0, _KB), pl.ds(coff, 128)], pr, sem)
        pltpu.async_copy(hp_hbm.at[sb, pl.ds(coff, 128)], mr, sem)

    def drain(pr, mr, sem):
        pltpu.make_async_copy(
            p_hbm.at[pl.ds(0, _KB), pl.ds(0, 128)], pr, sem).wait()
        pltpu.make_async_copy(
            p_hbm.at[pl.ds(0, _KB), pl.ds(0, 128)], mr, sem).wait()

    def mult(pr, mr):
        def edge(e, c2):
            for cc in range(8):
                pr[e, pl.ds(cc * 16, 16)] = (
                    pr[e, pl.ds(cc * 16, 16)] * mr[e, pl.ds(cc * 16, 16)])
            return c2

        lax.fori_loop(0, _KB, edge, 0, unroll=4)

    def scat_fire(i, pr, rb, sem):
        for k in range(_KB // 16):
            rb[0, pl.ds(k * 16, 16)] = lax.bitwise_and(
                packed[pl.ds(i * _KB + k * 16, 16)], 0xFFFF)
        pltpu.async_copy(pr, agg_sh.at[rb.at[0]], sem, add=True)

    def scat_drain(pr, rb, sem):
        pltpu.make_async_copy(pr, agg_sh.at[rb.at[0]], sem).wait()

    fire(0, pr0, mr0, sb0, sem0)
    fire(1, pr1, mr1, sb1, sem1)

    def pair(j, c2):
        b0 = 2 * j
        drain(pr0, mr0, sem0)
        mult(pr0, mr0)
        scat_fire(b0, pr0, rb0, ssem0)
        drain(pr1, mr1, sem1)
        mult(pr1, mr1)
        scat_fire(b0 + 1, pr1, rb1, ssem1)
        scat_drain(pr0, rb0, ssem0)

        @pl.when(b0 + 2 < _NBATCH)
        def _f0():
            fire(b0 + 2, pr0, mr0, sb0, sem0)

        scat_drain(pr1, rb1, ssem1)

        @pl.when(b0 + 3 < _NBATCH)
        def _f1():
            fire(b0 + 3, pr1, mr1, sb1, sem1)

        return c2

    lax.fori_loop(0, _NBATCH // 2, pair, 0)

    plsc.subcore_barrier()
    pltpu.sync_copy(
        agg_sh.at[pl.ds(row0, _RPT)],
        out_hbm.at[cid, pl.ds(row0, _RPT)],
    )

    @pl.when(sid == _NS - 1)
    def _out_tail():
        pltpu.sync_copy(
            agg_sh.at[pl.ds(row0 + _RPT, _ZR)],
            out_hbm.at[cid, pl.ds(row0 + _RPT, _ZR)],
        )


def _sc_scatter(hp, p, idx_packed):
    mesh = plsc.VectorSubcoreMesh(
        core_axis_name="c", subcore_axis_name="s",
        num_cores=_NC, num_subcores=_NS,
    )
    fn = functools.partial(
        pl.kernel,
        out_type=jax.ShapeDtypeStruct((_NC, _N, 128), jnp.float32),
        mesh=mesh,
        scratch_types=[
            pltpu.VMEM_SHARED((_N, 128), jnp.float32),
            pltpu.VMEM((_EPT,), jnp.int32),
            pltpu.VMEM((_KB,), jnp.int32),
            pltpu.VMEM((_KB,), jnp.int32),
            pltpu.VMEM((1, _KB), jnp.int32),
            pltpu.VMEM((1, _KB), jnp.int32),
            pltpu.VMEM((_KB, 128), jnp.float32),
            pltpu.VMEM((_KB, 128), jnp.float32),
            pltpu.VMEM((_KB, 128), jnp.float32),
            pltpu.VMEM((_KB, 128), jnp.float32),
            pltpu.VMEM((_ZR, 128), jnp.float32),
            pltpu.SemaphoreType.DMA,
            pltpu.SemaphoreType.DMA,
            pltpu.SemaphoreType.DMA,
            pltpu.SemaphoreType.DMA,
        ],
    )(_sc_body)
    return fn(hp, p, idx_packed)


def _post_body(a0_ref, a1_ref, nf_ref, wd0_ref, wd1_ref, wself_ref, wread_ref,
               outr_ref, outf_ref):
    a0 = a0_ref[:] * (1.0 / _AVG)
    a1 = a1_ref[:] * (1.0 / _AVG)
    ms = (jnp.sum(a0 * a0, axis=-1, keepdims=True)
          + jnp.sum(a1 * a1, axis=-1, keepdims=True)) * (1.0 / _MSG)
    inv = lax.rsqrt(ms + 1e-6)
    a0 = a0 * inv
    a1 = a1 * inv
    new = (jnp.dot(a0, wd0_ref[:], preferred_element_type=jnp.float32)
           + jnp.dot(a1, wd1_ref[:], preferred_element_type=jnp.float32))
    new = new * lax.rsqrt(jnp.mean(new * new, axis=-1, keepdims=True) + 1e-6)
    nfo = jnp.dot(nf_ref[:] + new, wself_ref[:],
                  preferred_element_type=jnp.float32)
    outf_ref[:] = nfo
    outr_ref[:] = jnp.dot(nfo, wread_ref[:],
                          preferred_element_type=jnp.float32)


def _post(a0, a1, nf, wd0, wd1, wself, wread):
    bn = 2000
    return pl.pallas_call(
        _post_body,
        grid=(_N // bn,),
        in_specs=[
            pl.BlockSpec((bn, 128), lambda i: (i, 0)),
            pl.BlockSpec((bn, 128), lambda i: (i, 0)),
            pl.BlockSpec((bn, _F), lambda i: (i, 0)),
            pl.BlockSpec((128, _F), lambda i: (0, 0)),
            pl.BlockSpec((128, _F), lambda i: (0, 0)),
            pl.BlockSpec((_F, _F), lambda i: (0, 0)),
            pl.BlockSpec((_F, 1), lambda i: (0, 0)),
        ],
        out_specs=[
            pl.BlockSpec((bn, 1), lambda i: (i, 0)),
            pl.BlockSpec((bn, _F), lambda i: (i, 0)),
        ],
        out_shape=[
            jax.ShapeDtypeStruct((_N, 1), jnp.float32),
            jax.ShapeDtypeStruct((_N, _F), jnp.float32),
        ],
    )(a0, a1, nf, wd0, wd1, wself, wread)


def kernel(vectors, node_feats, radial_embedding, senders, receivers,
           W_up, W_proj, W_rad, b_rad, W_down, W_self, W_read):
    wradp = W_rad[:, _PERM]
    bradp = b_rad[_PERM].reshape(1, _MSG)
    wdp = W_down[_PERM, :]
    wproj_exp = jnp.repeat(W_proj, 16, axis=1)
    kmask = jnp.asarray(_KMASK)
    w2_top = (wradp[:, None, :] * kmask[None, :, :]).reshape(_NB * 16, _MSG)
    w2f = jnp.concatenate([w2_top, bradp * kmask], axis=0)
    idx_packed = (lax.shift_left(senders.astype(jnp.int32), 16)
                  | receivers.astype(jnp.int32))
    idx_packed = jnp.pad(idx_packed, (0, _EPAD - _E))

    hp = _pre(node_feats, W_up, wproj_exp)
    vecT = jnp.pad(vectors.T, ((0, 0), (0, _EPAD - _E)))
    reT = jnp.pad(radial_embedding.T, ((0, 0), (0, _EPAD - _E)))
    p = _edge(vecT, reT, w2f)
    agg2 = _sc_scatter(hp, p, idx_packed)
    return _post(agg2[0], agg2[1], node_feats,
                 wdp[:128], wdp[128:], W_self, W_read)


# speedup vs baseline: 1.1513x; 1.1513x over previous
import functools

import jax
import jax.numpy as jnp
import numpy as np
from jax import lax
from jax.experimental import pallas as pl
from jax.experimental.pallas import tpu as pltpu
from jax.experimental.pallas import tpu_sc as plsc

_N = 10000
_E = 160000
_F = 128
_C = 16
_NB = 8
_MSG = 256
_AVG = 16.0

_PERM = np.empty(_MSG, np.int32)
for _j in range(_MSG):
    _c, _k = _j // 16, _j % 16
    _PERM[_j] = _c if _k == 0 else 16 + _c * 15 + (_k - 1)

_KMASK = (np.arange(16)[:, None] == (np.arange(_MSG)[None, :] % 16)).astype(
    np.float32)

_NC = 2
_NS = 16
_KB = 64
_EPAD = 163840
_EPT = _EPAD // _NS
_NBATCH = _EPT // _KB
_RPT = 624
_ZR = 16


def _pre_body(nf_ref, wup_ref, wproj_ref, hp_ref):
    h = jnp.dot(nf_ref[:], wup_ref[:], preferred_element_type=jnp.float32)
    h = h * lax.rsqrt(jnp.mean(h * h, axis=-1, keepdims=True) + 1e-6)
    hp_ref[:] = jnp.dot(h, wproj_ref[:], preferred_element_type=jnp.float32)


def _pre(nf, wup, wproj_exp):
    bn = 2000
    return pl.pallas_call(
        _pre_body,
        grid=(_N // bn,),
        in_specs=[
            pl.BlockSpec((bn, _F), lambda i: (i, 0)),
            pl.BlockSpec((_F, _F), lambda i: (0, 0)),
            pl.BlockSpec((_F, _F), lambda i: (0, 0)),
        ],
        out_specs=pl.BlockSpec((bn, _F), lambda i: (i, 0)),
        out_shape=jax.ShapeDtypeStruct((_N, _F), jnp.float32),
    )(nf, wup, wproj_exp)


def _edge_body(vt_ref, ret_ref, w2_ref, p_ref):
    x = vt_ref[0:1, :]
    y = vt_ref[1:2, :]
    z = vt_ref[2:3, :]
    r = lax.rsqrt(x * x + y * y + z * z + 1e-12)
    x = x * r
    y = y * r
    z = z * r
    s3 = 3.0 ** 0.5
    s5 = 5.0 ** 0.5
    s15 = 15.0 ** 0.5
    s358 = (35.0 / 8.0) ** 0.5
    s105 = 105.0 ** 0.5
    s218 = (21.0 / 8.0) ** 0.5
    s7 = 7.0 ** 0.5
    comps = [
        jnp.ones_like(x),
        s3 * x, s3 * y, s3 * z,
        s15 * x * y, s15 * y * z, (s5 / 2.0) * (3.0 * z * z - 1.0),
        s15 * x * z, (s15 / 2.0) * (x * x - y * y),
        s358 * y * (3.0 * x * x - y * y), s105 * x * y * z,
        s218 * y * (5.0 * z * z - 1.0),
        (s7 / 2.0) * (5.0 * z ** 3 - 3.0 * z),
        s218 * x * (5.0 * z * z - 1.0),
        (s105 / 2.0) * (x * x - y * y) * z,
        s358 * x * (x * x - 3.0 * y * y),
    ]
    be = x.shape[1]
    col = lax.broadcasted_iota(jnp.int32, (1, be), 1) + pl.program_id(0) * be
    valid = (col < _E).astype(jnp.float32)
    sT = jnp.concatenate(comps, axis=0) * valid
    blocks = [ret_ref[b:b + 1, :] * sT for b in range(_NB)]
    gT = jnp.concatenate(blocks + [sT], axis=0)
    p_ref[:] = lax.dot_general(
        gT, w2_ref[:], (((0,), (0,)), ((), ())),
        preferred_element_type=jnp.float32)


def _edge(vecT, reT, w2f):
    be = 2048
    return pl.pallas_call(
        _edge_body,
        grid=(_EPAD // be,),
        in_specs=[
            pl.BlockSpec((3, be), lambda i: (0, i)),
            pl.BlockSpec((_NB, be), lambda i: (0, i)),
            pl.BlockSpec((_NB * 16 + 16, _MSG), lambda i: (0, 0)),
        ],
        out_specs=pl.BlockSpec((be, _MSG), lambda i: (i, 0)),
        out_shape=jax.ShapeDtypeStruct((_EPAD, _MSG), jnp.float32),
    )(vecT, reT, w2f)


def _sc_body(hp_hbm, p_hbm, idx_hbm, out_hbm,
             agg_sh, packed, sb0, sb1, rb0, rb1, mr0, mr1, pr0, pr1, zbuf,
             sem0, sem1, ssem0, ssem1):
    cid = lax.axis_index("c")
    sid = lax.axis_index("s")
    is0 = cid == 0
    coff = cid * 128

    for rr in range(_ZR):
        for q in range(8):
            zbuf[rr, pl.ds(q * 16, 16)] = jnp.zeros((16,), jnp.float32)
    row0 = sid * _RPT
    for i in range(_RPT // _ZR):
        pltpu.sync_copy(zbuf, agg_sh.at[pl.ds(row0 + i * _ZR, _ZR)])

    @pl.when(sid == _NS - 1)
    def _zero_tail():
        pltpu.sync_copy(zbuf, agg_sh.at[pl.ds(row0 + _RPT, _ZR)])

    ebase = sid * _EPT
    pltpu.sync_copy(idx_hbm.at[pl.ds(ebase, _EPT)], packed)
    plsc.subcore_barrier()

    def fire(i, pr, mr, sb, sem):
        for k in range(_KB // 16):
            sb[pl.ds(k * 16, 16)] = lax.shift_right_logical(
                packed[pl.ds(i * _KB + k * 16, 16)], 16)
        e0 = ebase + i * _KB
        pltpu.async_copy(p_hbm.at[pl.ds(e0, _KB), pl.ds(coff, 128)], pr, sem)
        pltpu.async_copy(hp_hbm.at[sb], mr, sem)

    def drain(pr, mr, sem):
        pltpu.make_async_copy(
            p_hbm.at[pl.ds(0, _KB), pl.ds(0, 128)], pr, sem).wait()
        pltpu.make_async_copy(
            p_hbm.at[pl.ds(0, _KB), pl.ds(0, 128)], mr, sem).wait()

    lane0 = cid * 8

    def mult(pr, mr):
        def edge(e, c2):
            mv = mr[e, pl.ds(lane0, 16)]
            for cc in range(8):
                pr[e, pl.ds(cc * 16, 16)] = pr[e, pl.ds(cc * 16, 16)] * mv[cc]
            return c2

        lax.fori_loop(0, _KB, edge, 0, unroll=2)

    def scat_fire(i, pr, rb, sem):
        for k in range(_KB // 16):
            rb[0, pl.ds(k * 16, 16)] = lax.bitwise_and(
                packed[pl.ds(i * _KB + k * 16, 16)], 0xFFFF)
        pltpu.async_copy(pr, agg_sh.at[rb.at[0]], sem, add=True)

    def scat_drain(pr, rb, sem):
        pltpu.make_async_copy(pr, agg_sh.at[rb.at[0]], sem).wait()

    fire(0, pr0, mr0, sb0, sem0)
    fire(1, pr1, mr1, sb1, sem1)

    def pair(j, c2):
        b0 = 2 * j
        drain(pr0, mr0, sem0)
        mult(pr0, mr0)
        scat_fire(b0, pr0, rb0, ssem0)
        drain(pr1, mr1, sem1)
        mult(pr1, mr1)
        scat_fire(b0 + 1, pr1, rb1, ssem1)
        scat_drain(pr0, rb0, ssem0)

        @pl.when(b0 + 2 < _NBATCH)
        def _f0():
            fire(b0 + 2, pr0, mr0, sb0, sem0)

        scat_drain(pr1, rb1, ssem1)

        @pl.when(b0 + 3 < _NBATCH)
        def _f1():
            fire(b0 + 3, pr1, mr1, sb1, sem1)

        return c2

    lax.fori_loop(0, _NBATCH // 2, pair, 0)

    plsc.subcore_barrier()
    pltpu.sync_copy(
        agg_sh.at[pl.ds(row0, _RPT)],
        out_hbm.at[cid, pl.ds(row0, _RPT)],
    )

    @pl.when(sid == _NS - 1)
    def _out_tail():
        pltpu.sync_copy(
            agg_sh.at[pl.ds(row0 + _RPT, _ZR)],
            out_hbm.at[cid, pl.ds(row0 + _RPT, _ZR)],
        )


def _sc_scatter(hp, p, idx_packed):
    mesh = plsc.VectorSubcoreMesh(
        core_axis_name="c", subcore_axis_name="s",
        num_cores=_NC, num_subcores=_NS,
    )
    fn = functools.partial(
        pl.kernel,
        out_type=jax.ShapeDtypeStruct((_NC, _N, 128), jnp.float32),
        mesh=mesh,
        scratch_types=[
            pltpu.VMEM_SHARED((_N, 128), jnp.float32),
            pltpu.VMEM((_EPT,), jnp.int32),
            pltpu.VMEM((_KB,), jnp.int32),
            pltpu.VMEM((_KB,), jnp.int32),
            pltpu.VMEM((1, _KB), jnp.int32),
            pltpu.VMEM((1, _KB), jnp.int32),
            pltpu.VMEM((_KB, 128), jnp.float32),
            pltpu.VMEM((_KB, 128), jnp.float32),
            pltpu.VMEM((_KB, 128), jnp.float32),
            pltpu.VMEM((_KB, 128), jnp.float32),
            pltpu.VMEM((_ZR, 128), jnp.float32),
            pltpu.SemaphoreType.DMA,
            pltpu.SemaphoreType.DMA,
            pltpu.SemaphoreType.DMA,
            pltpu.SemaphoreType.DMA,
        ],
    )(_sc_body)
    return fn(hp, p, idx_packed)


def _post_body(a0_ref, a1_ref, nf_ref, wd0_ref, wd1_ref, wself_ref, wread_ref,
               outr_ref, outf_ref):
    a0 = a0_ref[:] * (1.0 / _AVG)
    a1 = a1_ref[:] * (1.0 / _AVG)
    ms = (jnp.sum(a0 * a0, axis=-1, keepdims=True)
          + jnp.sum(a1 * a1, axis=-1, keepdims=True)) * (1.0 / _MSG)
    inv = lax.rsqrt(ms + 1e-6)
    a0 = a0 * inv
    a1 = a1 * inv
    new = (jnp.dot(a0, wd0_ref[:], preferred_element_type=jnp.float32)
           + jnp.dot(a1, wd1_ref[:], preferred_element_type=jnp.float32))
    new = new * lax.rsqrt(jnp.mean(new * new, axis=-1, keepdims=True) + 1e-6)
    nfo = jnp.dot(nf_ref[:] + new, wself_ref[:],
                  preferred_element_type=jnp.float32)
    outf_ref[:] = nfo
    outr_ref[:] = jnp.dot(nfo, wread_ref[:],
                          preferred_element_type=jnp.float32)


def _post(a0, a1, nf, wd0, wd1, wself, wread):
    bn = 2000
    return pl.pallas_call(
        _post_body,
        grid=(_N // bn,),
        in_specs=[
            pl.BlockSpec((bn, 128), lambda i: (i, 0)),
            pl.BlockSpec((bn, 128), lambda i: (i, 0)),
            pl.BlockSpec((bn, _F), lambda i: (i, 0)),
            pl.BlockSpec((128, _F), lambda i: (0, 0)),
            pl.BlockSpec((128, _F), lambda i: (0, 0)),
            pl.BlockSpec((_F, _F), lambda i: (0, 0)),
            pl.BlockSpec((_F, 1), lambda i: (0, 0)),
        ],
        out_specs=[
            pl.BlockSpec((bn, 1), lambda i: (i, 0)),
            pl.BlockSpec((bn, _F), lambda i: (i, 0)),
        ],
        out_shape=[
            jax.ShapeDtypeStruct((_N, 1), jnp.float32),
            jax.ShapeDtypeStruct((_N, _F), jnp.float32),
        ],
    )(a0, a1, nf, wd0, wd1, wself, wread)


def kernel(vectors, node_feats, radial_embedding, senders, receivers,
           W_up, W_proj, W_rad, b_rad, W_down, W_self, W_read):
    wradp = W_rad[:, _PERM]
    bradp = b_rad[_PERM].reshape(1, _MSG)
    wdp = W_down[_PERM, :]
    wproj_exp = jnp.pad(W_proj, ((0, 0), (0, _F - _C)))
    kmask = jnp.asarray(_KMASK)
    w2_top = (wradp[:, None, :] * kmask[None, :, :]).reshape(_NB * 16, _MSG)
    w2f = jnp.concatenate([w2_top, bradp * kmask], axis=0)
    idx_packed = (lax.shift_left(senders.astype(jnp.int32), 16)
                  | receivers.astype(jnp.int32))
    idx_packed = jnp.pad(idx_packed, (0, _EPAD - _E))

    hp = _pre(node_feats, W_up, wproj_exp)
    vecT = jnp.pad(vectors.T, ((0, 0), (0, _EPAD - _E)))
    reT = jnp.pad(radial_embedding.T, ((0, 0), (0, _EPAD - _E)))
    p = _edge(vecT, reT, w2f)
    agg2 = _sc_scatter(hp, p, idx_packed)
    return _post(agg2[0], agg2[1], node_feats,
                 wdp[:128], wdp[128:], W_self, W_read)


# speedup vs baseline: 1.1837x; 1.0281x over previous
import functools

import jax
import jax.numpy as jnp
import numpy as np
from jax import lax
from jax.experimental import pallas as pl
from jax.experimental.pallas import tpu as pltpu
from jax.experimental.pallas import tpu_sc as plsc

_N = 10000
_E = 160000
_F = 128
_C = 16
_NB = 8
_MSG = 256
_AVG = 16.0

_PERM = np.empty(_MSG, np.int32)
for _j in range(_MSG):
    _c, _k = _j // 16, _j % 16
    _PERM[_j] = _c if _k == 0 else 16 + _c * 15 + (_k - 1)

_KMASK = (np.arange(16)[:, None] == (np.arange(_MSG)[None, :] % 16)).astype(
    np.float32)

_NC = 2
_NS = 16
_KB = 64
_EPAD = 163840
_EPT = _EPAD // _NS
_NBATCH = _EPT // _KB
_RPT = 624
_ZR = 16


def _pre_body(nf_ref, wup_ref, wproj_ref, hp_ref):
    h = jnp.dot(nf_ref[:], wup_ref[:], preferred_element_type=jnp.float32)
    h = h * lax.rsqrt(jnp.mean(h * h, axis=-1, keepdims=True) + 1e-6)
    hp_ref[:] = jnp.dot(h, wproj_ref[:], preferred_element_type=jnp.float32)


def _pre(nf, wup, wproj_exp):
    bn = 2000
    return pl.pallas_call(
        _pre_body,
        grid=(_N // bn,),
        in_specs=[
            pl.BlockSpec((bn, _F), lambda i: (i, 0)),
            pl.BlockSpec((_F, _F), lambda i: (0, 0)),
            pl.BlockSpec((_F, _F), lambda i: (0, 0)),
        ],
        out_specs=pl.BlockSpec((bn, _F), lambda i: (i, 0)),
        out_shape=jax.ShapeDtypeStruct((_N, _F), jnp.float32),
    )(nf, wup, wproj_exp)


def _edge_body(vt_ref, ret_ref, w2_ref, p_ref):
    x = vt_ref[0:1, :]
    y = vt_ref[1:2, :]
    z = vt_ref[2:3, :]
    r = lax.rsqrt(x * x + y * y + z * z + 1e-12)
    x = x * r
    y = y * r
    z = z * r
    s3 = 3.0 ** 0.5
    s5 = 5.0 ** 0.5
    s15 = 15.0 ** 0.5
    s358 = (35.0 / 8.0) ** 0.5
    s105 = 105.0 ** 0.5
    s218 = (21.0 / 8.0) ** 0.5
    s7 = 7.0 ** 0.5
    comps = [
        jnp.ones_like(x),
        s3 * x, s3 * y, s3 * z,
        s15 * x * y, s15 * y * z, (s5 / 2.0) * (3.0 * z * z - 1.0),
        s15 * x * z, (s15 / 2.0) * (x * x - y * y),
        s358 * y * (3.0 * x * x - y * y), s105 * x * y * z,
        s218 * y * (5.0 * z * z - 1.0),
        (s7 / 2.0) * (5.0 * z ** 3 - 3.0 * z),
        s218 * x * (5.0 * z * z - 1.0),
        (s105 / 2.0) * (x * x - y * y) * z,
        s358 * x * (x * x - 3.0 * y * y),
    ]
    be = x.shape[1]
    col = lax.broadcasted_iota(jnp.int32, (1, be), 1) + pl.program_id(0) * be
    valid = (col < _E).astype(jnp.float32)
    sT = jnp.concatenate(comps, axis=0) * valid
    blocks = [ret_ref[b:b + 1, :] * sT for b in range(_NB)]
    gT = jnp.concatenate(blocks + [sT], axis=0)
    p_ref[:] = lax.dot_general(
        gT, w2_ref[:], (((0,), (0,)), ((), ())),
        preferred_element_type=jnp.float32)


def _edge(vecT, reT, w2f):
    be = 2048
    return pl.pallas_call(
        _edge_body,
        grid=(_EPAD // be,),
        in_specs=[
            pl.BlockSpec((3, be), lambda i: (0, i)),
            pl.BlockSpec((_NB, be), lambda i: (0, i)),
            pl.BlockSpec((_NB * 16 + 16, _MSG), lambda i: (0, 0)),
        ],
        out_specs=pl.BlockSpec((be, _MSG), lambda i: (i, 0)),
        out_shape=jax.ShapeDtypeStruct((_EPAD, _MSG), jnp.float32),
    )(vecT, reT, w2f)


def _sc_body(hp_hbm, p_hbm, idx_hbm, out_hbm,
             agg_sh, packed, sb0, sb1, rb0, rb1, mr0, mr1, pr0, pr1, zbuf,
             gsem0, gsem1, psem0, psem1, ssem0, ssem1):
    cid = lax.axis_index("c")
    sid = lax.axis_index("s")
    is0 = cid == 0
    coff = cid * 128

    for rr in range(_ZR):
        for q in range(8):
            zbuf[rr, pl.ds(q * 16, 16)] = jnp.zeros((16,), jnp.float32)
    row0 = sid * _RPT
    for i in range(_RPT // _ZR):
        pltpu.sync_copy(zbuf, agg_sh.at[pl.ds(row0 + i * _ZR, _ZR)])

    @pl.when(sid == _NS - 1)
    def _zero_tail():
        pltpu.sync_copy(zbuf, agg_sh.at[pl.ds(row0 + _RPT, _ZR)])

    ebase = sid * _EPT
    pltpu.sync_copy(idx_hbm.at[pl.ds(ebase, _EPT)], packed)
    plsc.subcore_barrier()

    def fire_g(i, mr, sb, sem):
        for k in range(_KB // 16):
            sb[pl.ds(k * 16, 16)] = lax.shift_right_logical(
                packed[pl.ds(i * _KB + k * 16, 16)], 16)
        pltpu.async_copy(hp_hbm.at[sb], mr, sem)

    def fire_p(i, pr, sem):
        e0 = ebase + i * _KB
        pltpu.async_copy(p_hbm.at[pl.ds(e0, _KB), pl.ds(coff, 128)], pr, sem)

    def drain(pr, mr, gsem, psem):
        pltpu.make_async_copy(
            p_hbm.at[pl.ds(0, _KB), pl.ds(0, 128)], mr, gsem).wait()
        pltpu.make_async_copy(
            p_hbm.at[pl.ds(0, _KB), pl.ds(0, 128)], pr, psem).wait()

    lane0 = cid * 8

    def mult(pr, mr):
        def edge(e, c2):
            mv = mr[e, pl.ds(lane0, 16)]
            for cc in range(8):
                pr[e, pl.ds(cc * 16, 16)] = pr[e, pl.ds(cc * 16, 16)] * mv[cc]
            return c2

        lax.fori_loop(0, _KB, edge, 0, unroll=2)

    def scat_fire(i, pr, rb, sem):
        for k in range(_KB // 16):
            rb[0, pl.ds(k * 16, 16)] = lax.bitwise_and(
                packed[pl.ds(i * _KB + k * 16, 16)], 0xFFFF)
        pltpu.async_copy(pr, agg_sh.at[rb.at[0]], sem, add=True)

    def scat_drain(pr, rb, sem):
        pltpu.make_async_copy(pr, agg_sh.at[rb.at[0]], sem).wait()

    fire_g(0, mr0, sb0, gsem0)
    fire_p(0, pr0, psem0)
    fire_g(1, mr1, sb1, gsem1)
    fire_p(1, pr1, psem1)

    def pair(j, c2):
        b0 = 2 * j
        drain(pr0, mr0, gsem0, psem0)
        mult(pr0, mr0)

        @pl.when(b0 + 2 < _NBATCH)
        def _g0():
            fire_g(b0 + 2, mr0, sb0, gsem0)

        scat_fire(b0, pr0, rb0, ssem0)
        drain(pr1, mr1, gsem1, psem1)
        mult(pr1, mr1)

        @pl.when(b0 + 3 < _NBATCH)
        def _g1():
            fire_g(b0 + 3, mr1, sb1, gsem1)

        scat_fire(b0 + 1, pr1, rb1, ssem1)
        scat_drain(pr0, rb0, ssem0)

        @pl.when(b0 + 2 < _NBATCH)
        def _p0():
            fire_p(b0 + 2, pr0, psem0)

        scat_drain(pr1, rb1, ssem1)

        @pl.when(b0 + 3 < _NBATCH)
        def _p1():
            fire_p(b0 + 3, pr1, psem1)

        return c2

    lax.fori_loop(0, _NBATCH // 2, pair, 0)

    plsc.subcore_barrier()
    pltpu.sync_copy(
        agg_sh.at[pl.ds(row0, _RPT)],
        out_hbm.at[cid, pl.ds(row0, _RPT)],
    )

    @pl.when(sid == _NS - 1)
    def _out_tail():
        pltpu.sync_copy(
            agg_sh.at[pl.ds(row0 + _RPT, _ZR)],
            out_hbm.at[cid, pl.ds(row0 + _RPT, _ZR)],
        )


def _sc_scatter(hp, p, idx_packed):
    mesh = plsc.VectorSubcoreMesh(
        core_axis_name="c", subcore_axis_name="s",
        num_cores=_NC, num_subcores=_NS,
    )
    fn = functools.partial(
        pl.kernel,
        out_type=jax.ShapeDtypeStruct((_NC, _N, 128), jnp.float32),
        mesh=mesh,
        scratch_types=[
            pltpu.VMEM_SHARED((_N, 128), jnp.float32),
            pltpu.VMEM((_EPT,), jnp.int32),
            pltpu.VMEM((_KB,), jnp.int32),
            pltpu.VMEM((_KB,), jnp.int32),
            pltpu.VMEM((1, _KB), jnp.int32),
            pltpu.VMEM((1, _KB), jnp.int32),
            pltpu.VMEM((_KB, 128), jnp.float32),
            pltpu.VMEM((_KB, 128), jnp.float32),
            pltpu.VMEM((_KB, 128), jnp.float32),
            pltpu.VMEM((_KB, 128), jnp.float32),
            pltpu.VMEM((_ZR, 128), jnp.float32),
            pltpu.SemaphoreType.DMA,
            pltpu.SemaphoreType.DMA,
            pltpu.SemaphoreType.DMA,
            pltpu.SemaphoreType.DMA,
            pltpu.SemaphoreType.DMA,
            pltpu.SemaphoreType.DMA,
        ],
    )(_sc_body)
    return fn(hp, p, idx_packed)


def _post_body(a0_ref, a1_ref, nf_ref, wd0_ref, wd1_ref, wself_ref, wread_ref,
               outr_ref, outf_ref):
    a0 = a0_ref[:] * (1.0 / _AVG)
    a1 = a1_ref[:] * (1.0 / _AVG)
    ms = (jnp.sum(a0 * a0, axis=-1, keepdims=True)
          + jnp.sum(a1 * a1, axis=-1, keepdims=True)) * (1.0 / _MSG)
    inv = lax.rsqrt(ms + 1e-6)
    a0 = a0 * inv
    a1 = a1 * inv
    new = (jnp.dot(a0, wd0_ref[:], preferred_element_type=jnp.float32)
           + jnp.dot(a1, wd1_ref[:], preferred_element_type=jnp.float32))
    new = new * lax.rsqrt(jnp.mean(new * new, axis=-1, keepdims=True) + 1e-6)
    nfo = jnp.dot(nf_ref[:] + new, wself_ref[:],
                  preferred_element_type=jnp.float32)
    outf_ref[:] = nfo
    outr_ref[:] = jnp.dot(nfo, wread_ref[:],
                          preferred_element_type=jnp.float32)


def _post(a0, a1, nf, wd0, wd1, wself, wread):
    bn = 2000
    return pl.pallas_call(
        _post_body,
        grid=(_N // bn,),
        in_specs=[
            pl.BlockSpec((bn, 128), lambda i: (i, 0)),
            pl.BlockSpec((bn, 128), lambda i: (i, 0)),
            pl.BlockSpec((bn, _F), lambda i: (i, 0)),
            pl.BlockSpec((128, _F), lambda i: (0, 0)),
            pl.BlockSpec((128, _F), lambda i: (0, 0)),
            pl.BlockSpec((_F, _F), lambda i: (0, 0)),
            pl.BlockSpec((_F, 1), lambda i: (0, 0)),
        ],
        out_specs=[
            pl.BlockSpec((bn, 1), lambda i: (i, 0)),
            pl.BlockSpec((bn, _F), lambda i: (i, 0)),
        ],
        out_shape=[
            jax.ShapeDtypeStruct((_N, 1), jnp.float32),
            jax.ShapeDtypeStruct((_N, _F), jnp.float32),
        ],
    )(a0, a1, nf, wd0, wd1, wself, wread)


def kernel(vectors, node_feats, radial_embedding, senders, receivers,
           W_up, W_proj, W_rad, b_rad, W_down, W_self, W_read):
    wradp = W_rad[:, _PERM]
    bradp = b_rad[_PERM].reshape(1, _MSG)
    wdp = W_down[_PERM, :]
    wproj_exp = jnp.pad(W_proj, ((0, 0), (0, _F - _C)))
    kmask = jnp.asarray(_KMASK)
    w2_top = (wradp[:, None, :] * kmask[None, :, :]).reshape(_NB * 16, _MSG)
    w2f = jnp.concatenate([w2_top, bradp * kmask], axis=0)
    idx_packed = (lax.shift_left(senders.astype(jnp.int32), 16)
                  | receivers.astype(jnp.int32))
    idx_packed = jnp.pad(idx_packed, (0, _EPAD - _E))

    hp = _pre(node_feats, W_up, wproj_exp)
    vecT = jnp.pad(vectors.T, ((0, 0), (0, _EPAD - _E)))
    reT = jnp.pad(radial_embedding.T, ((0, 0), (0, _EPAD - _E)))
    p = _edge(vecT, reT, w2f)
    agg2 = _sc_scatter(hp, p, idx_packed)
    return _post(agg2[0], agg2[1], node_feats,
                 wdp[:128], wdp[128:], W_self, W_read)
